# Initial kernel scaffold; baseline (speedup 1.0000x reference)
import jax
import jax.numpy as jnp
from jax.experimental import pallas as pl


def kernel(x, edge_index, W):
    raise NotImplementedError("write your pallas kernel here")


# speedup vs baseline: 42.8558x; 42.8558x over previous
import functools

import jax
import jax.numpy as jnp
import numpy as np
from jax import lax
from jax.experimental import pallas as pl
from jax.experimental.pallas import tpu as pltpu
from jax.experimental.pallas import tpu_sc as plsc

N_NODES = 10000
D_FEAT = 128
N_EDGES = 320000

NC = 2
NS = 16
DH = D_FEAT // 2
CHUNK = 128
CPT = 80
E_PAD = NC * NS * CPT * CHUNK
R_ACC = 10112
ROWS_PER_TILE = R_ACC // NS
DEG_W = 8

_HARM = np.cumsum((1.0 / np.arange(1, 8)).astype(np.float32), dtype=np.float32)


def _sc_segment_sum(x_lo, x_hi, rows3, cols3, zsum, zdeg, onescol):
    mesh = plsc.VectorSubcoreMesh(core_axis_name="c", subcore_axis_name="s")

    @functools.partial(
        pl.kernel,
        out_type=[
            jax.ShapeDtypeStruct((NC, R_ACC, DH), jnp.float32),
            jax.ShapeDtypeStruct((NC, R_ACC, DH), jnp.float32),
            jax.ShapeDtypeStruct((NC, R_ACC, DEG_W), jnp.float32),
        ],
        mesh=mesh,
        scratch_types=[
            pltpu.VMEM((CPT, CHUNK), jnp.int32),
            pltpu.VMEM((CPT, CHUNK), jnp.int32),
            pltpu.VMEM((CHUNK, DH), jnp.float32),
            pltpu.VMEM((CHUNK, DEG_W), jnp.float32),
            pltpu.VMEM_SHARED((R_ACC, DH), jnp.float32),
            pltpu.VMEM_SHARED((R_ACC, DEG_W), jnp.float32),
            pltpu.SemaphoreType.DMA,
        ],
        compiler_params=pltpu.CompilerParams(use_tc_tiling_on_sc=False),
    )
    def k(xlo_hbm, xhi_hbm, rows_hbm, cols_hbm, zsum_hbm, zdeg_hbm, ones_hbm,
          sumlo_out, sumhi_out, deg_out,
          colidx_v, rowidx_v, rows_v, ones_v, sum_sh, deg_sh, sem):
        c = lax.axis_index("c")
        s = lax.axis_index("s")
        wid = c * NS + s
        base = s * ROWS_PER_TILE

        pltpu.sync_copy(ones_hbm, ones_v)
        pltpu.sync_copy(cols_hbm.at[wid], colidx_v)
        pltpu.sync_copy(rows_hbm.at[wid], rowidx_v)

        for p, (x_hbm, out_ref) in enumerate(
                [(xlo_hbm, sumlo_out), (xhi_hbm, sumhi_out)]):
            pltpu.sync_copy(zsum_hbm, sum_sh.at[pl.ds(base, ROWS_PER_TILE)])
            if p == 0:
                pltpu.sync_copy(zdeg_hbm, deg_sh.at[pl.ds(base, ROWS_PER_TILE)])
            plsc.subcore_barrier()

            def body(j, carry):
                pltpu.async_copy(x_hbm.at[colidx_v.at[j]], rows_v, sem).wait()
                pltpu.sync_copy(rows_v, sum_sh.at[rowidx_v.at[j]], add=True)
                if p == 0:
                    pltpu.sync_copy(ones_v, deg_sh.at[rowidx_v.at[j]], add=True)
                return carry

            lax.fori_loop(jnp.int32(0), jnp.int32(CPT), body, jnp.int32(0))
            plsc.subcore_barrier()

            pltpu.sync_copy(sum_sh.at[pl.ds(base, ROWS_PER_TILE)],
                            out_ref.at[c, pl.ds(base, ROWS_PER_TILE)])
            if p == 0:
                pltpu.sync_copy(deg_sh.at[pl.ds(base, ROWS_PER_TILE)],
                                deg_out.at[c, pl.ds(base, ROWS_PER_TILE)])

    return k(x_lo, x_hi, rows3, cols3, zsum, zdeg, onescol)


def _i0():
    return jnp.int32(0)


def _tc_combine_kernel(x_ref, wt_ref, sumlo_ref, sumhi_ref, deg_ref, out_ref):
    x = x_ref[...]
    sumn = jnp.concatenate(
        [sumlo_ref[0] + sumlo_ref[1], sumhi_ref[0] + sumhi_ref[1]], axis=1)
    degw = deg_ref[0] + deg_ref[1]
    degc = degw[:, 0:1]
    safe = jnp.maximum(degc, 1.0)
    mean = sumn / safe
    h = _HARM
    hd1 = jnp.where(
        degc < 1.5, jnp.where(degc < 0.5, h[0], h[1]),
        jnp.where(degc < 3.5,
                  jnp.where(degc < 2.5, h[2], h[3]),
                  jnp.where(degc < 4.5, h[4],
                            jnp.where(degc < 5.5, h[5], h[6]))))
    exact = (x * hd1 - mean * (hd1 - 1.0)) / (degc + 1.0)
    contrib = jnp.where(degc < 0.5, jnp.zeros_like(x),
                        jnp.where(degc < 5.5, exact, mean))
    shap = x + contrib
    acc = jnp.dot(shap, wt_ref[...], preferred_element_type=jnp.float32,
                  precision=lax.Precision.HIGHEST)
    out_ref[...] = jnp.maximum(acc, 0.0)


def _tc_combine(x, wt, sum_lo, sum_hi, deg_partial):
    br = 1000
    grid = (N_NODES // br,)
    return pl.pallas_call(
        _tc_combine_kernel,
        grid=grid,
        in_specs=[
            pl.BlockSpec((br, D_FEAT), lambda i: (i, _i0())),
            pl.BlockSpec((D_FEAT, D_FEAT), lambda i: (_i0(), _i0())),
            pl.BlockSpec((NC, br, DH), lambda i: (_i0(), i, _i0())),
            pl.BlockSpec((NC, br, DH), lambda i: (_i0(), i, _i0())),
            pl.BlockSpec((NC, br, DEG_W), lambda i: (_i0(), i, _i0())),
        ],
        out_specs=pl.BlockSpec((br, D_FEAT), lambda i: (i, _i0())),
        out_shape=jax.ShapeDtypeStruct((N_NODES, D_FEAT), jnp.float32),
    )(x, wt, sum_lo, sum_hi, deg_partial)


def kernel(x, edge_index, W):
    x = x.astype(jnp.float32)
    row = edge_index[0].astype(jnp.int32)
    col = edge_index[1].astype(jnp.int32)
    n_pad = E_PAD - N_EDGES
    rows3 = jnp.concatenate(
        [row, jnp.full((n_pad,), N_NODES, jnp.int32)]).reshape(NC * NS, CPT, CHUNK)
    cols3 = jnp.concatenate(
        [col, jnp.zeros((n_pad,), jnp.int32)]).reshape(NC * NS, CPT, CHUNK)
    x_lo = x[:, :DH]
    x_hi = x[:, DH:]
    zsum = jnp.zeros((ROWS_PER_TILE, DH), jnp.float32)
    zdeg = jnp.zeros((ROWS_PER_TILE, DEG_W), jnp.float32)
    onescol = jnp.zeros((CHUNK, DEG_W), jnp.float32).at[:, 0].set(1.0)

    sum_lo, sum_hi, deg_partial = _sc_segment_sum(
        x_lo, x_hi, rows3, cols3, zsum, zdeg, onescol)
    wt = W.astype(jnp.float32).T
    return _tc_combine(x, wt, sum_lo, sum_hi, deg_partial).astype(jnp.float64)


# speedup vs baseline: 46.6009x; 1.0874x over previous
import functools

import jax
import jax.numpy as jnp
import numpy as np
from jax import lax
from jax.experimental import pallas as pl
from jax.experimental.pallas import tpu as pltpu
from jax.experimental.pallas import tpu_sc as plsc

N_NODES = 10000
D_FEAT = 128
N_EDGES = 320000

NC = 2
NS = 16
DH = D_FEAT // 2
CHUNK = 128
CPT = 80
E_PAD = NC * NS * CPT * CHUNK
R_ACC = 10112
ROWS_PER_TILE = R_ACC // NS
DEG_W = 8

_HARM = np.cumsum((1.0 / np.arange(1, 8)).astype(np.float32), dtype=np.float32)


def _sc_segment_sum(x_lo, x_hi, rows3, cols3, zsum, zdeg, onescol):
    mesh = plsc.VectorSubcoreMesh(core_axis_name="c", subcore_axis_name="s")

    @functools.partial(
        pl.kernel,
        out_type=[
            jax.ShapeDtypeStruct((NC, R_ACC, DH), jnp.float32),
            jax.ShapeDtypeStruct((NC, R_ACC, DH), jnp.float32),
            jax.ShapeDtypeStruct((NC, R_ACC, DEG_W), jnp.float32),
        ],
        mesh=mesh,
        scratch_types=[
            pltpu.VMEM((CPT, CHUNK), jnp.int32),
            pltpu.VMEM((CPT, CHUNK), jnp.int32),
            pltpu.VMEM((CHUNK, DH), jnp.float32),
            pltpu.VMEM((CHUNK, DH), jnp.float32),
            pltpu.VMEM((CHUNK, DEG_W), jnp.float32),
            pltpu.VMEM_SHARED((R_ACC, DH), jnp.float32),
            pltpu.VMEM_SHARED((R_ACC, DEG_W), jnp.float32),
            pltpu.SemaphoreType.DMA,
            pltpu.SemaphoreType.DMA,
        ],
        compiler_params=pltpu.CompilerParams(use_tc_tiling_on_sc=False),
    )
    def k(xlo_hbm, xhi_hbm, rows_hbm, cols_hbm, zsum_hbm, zdeg_hbm, ones_hbm,
          sumlo_out, sumhi_out, deg_out,
          colidx_v, rowidx_v, rows_v0, rows_v1, ones_v, sum_sh, deg_sh,
          sem, dsem):
        c = lax.axis_index("c")
        s = lax.axis_index("s")
        wid = c * NS + s
        base = s * ROWS_PER_TILE
        bufs = (rows_v0, rows_v1)

        pltpu.sync_copy(ones_hbm, ones_v)
        pltpu.sync_copy(cols_hbm.at[wid], colidx_v)
        pltpu.sync_copy(rows_hbm.at[wid], rowidx_v)

        for p, (x_hbm, out_ref) in enumerate(
                [(xlo_hbm, sumlo_out), (xhi_hbm, sumhi_out)]):
            pltpu.sync_copy(zsum_hbm, sum_sh.at[pl.ds(base, ROWS_PER_TILE)])
            if p == 0:
                pltpu.sync_copy(zdeg_hbm, deg_sh.at[pl.ds(base, ROWS_PER_TILE)])
            plsc.subcore_barrier()

            pltpu.async_copy(x_hbm.at[colidx_v.at[jnp.int32(0)]], bufs[0], sem)

            def body(t, carry):
                for b in range(2):
                    j = t * 2 + b
                    cur, nxt = bufs[b], bufs[1 - b]
                    pltpu.make_async_copy(
                        x_hbm.at[colidx_v.at[j]], cur, sem).wait()
                    nj = j + 1

                    @pl.when(nj < CPT)
                    def _():
                        pltpu.async_copy(x_hbm.at[colidx_v.at[nj]], nxt, sem)

                    pltpu.sync_copy(cur, sum_sh.at[rowidx_v.at[j]], add=True)
                    if p == 0:
                        pltpu.async_copy(
                            ones_v, deg_sh.at[rowidx_v.at[j]], dsem, add=True)
                return carry

            lax.fori_loop(jnp.int32(0), jnp.int32(CPT // 2), body, jnp.int32(0))
            if p == 0:
                def drain(j, carry):
                    pltpu.make_async_copy(
                        ones_v, deg_sh.at[rowidx_v.at[j]], dsem).wait()
                    return carry

                lax.fori_loop(jnp.int32(0), jnp.int32(CPT), drain, jnp.int32(0))
            plsc.subcore_barrier()

            pltpu.sync_copy(sum_sh.at[pl.ds(base, ROWS_PER_TILE)],
                            out_ref.at[c, pl.ds(base, ROWS_PER_TILE)])
            if p == 0:
                pltpu.sync_copy(deg_sh.at[pl.ds(base, ROWS_PER_TILE)],
                                deg_out.at[c, pl.ds(base, ROWS_PER_TILE)])

    return k(x_lo, x_hi, rows3, cols3, zsum, zdeg, onescol)


def _i0():
    return jnp.int32(0)


def _tc_combine_kernel(x_ref, wt_ref, sumlo_ref, sumhi_ref, deg_ref, out_ref):
    x = x_ref[...]
    sumn = jnp.concatenate(
        [sumlo_ref[0] + sumlo_ref[1], sumhi_ref[0] + sumhi_ref[1]], axis=1)
    degw = deg_ref[0] + deg_ref[1]
    degc = degw[:, 0:1]
    safe = jnp.maximum(degc, 1.0)
    mean = sumn / safe
    h = _HARM
    hd1 = jnp.where(
        degc < 1.5, jnp.where(degc < 0.5, h[0], h[1]),
        jnp.where(degc < 3.5,
                  jnp.where(degc < 2.5, h[2], h[3]),
                  jnp.where(degc < 4.5, h[4],
                            jnp.where(degc < 5.5, h[5], h[6]))))
    exact = (x * hd1 - mean * (hd1 - 1.0)) / (degc + 1.0)
    contrib = jnp.where(degc < 0.5, jnp.zeros_like(x),
                        jnp.where(degc < 5.5, exact, mean))
    shap = x + contrib
    acc = jnp.dot(shap, wt_ref[...], preferred_element_type=jnp.float32,
                  precision=lax.Precision.HIGHEST)
    out_ref[...] = jnp.maximum(acc, 0.0)


def _tc_combine(x, wt, sum_lo, sum_hi, deg_partial):
    br = 1000
    grid = (N_NODES // br,)
    return pl.pallas_call(
        _tc_combine_kernel,
        grid=grid,
        in_specs=[
            pl.BlockSpec((br, D_FEAT), lambda i: (i, _i0())),
            pl.BlockSpec((D_FEAT, D_FEAT), lambda i: (_i0(), _i0())),
            pl.BlockSpec((NC, br, DH), lambda i: (_i0(), i, _i0())),
            pl.BlockSpec((NC, br, DH), lambda i: (_i0(), i, _i0())),
            pl.BlockSpec((NC, br, DEG_W), lambda i: (_i0(), i, _i0())),
        ],
        out_specs=pl.BlockSpec((br, D_FEAT), lambda i: (i, _i0())),
        out_shape=jax.ShapeDtypeStruct((N_NODES, D_FEAT), jnp.float32),
    )(x, wt, sum_lo, sum_hi, deg_partial)


def kernel(x, edge_index, W):
    x = x.astype(jnp.float32)
    row = edge_index[0].astype(jnp.int32)
    col = edge_index[1].astype(jnp.int32)
    n_pad = E_PAD - N_EDGES
    rows3 = jnp.concatenate(
        [row, jnp.full((n_pad,), N_NODES, jnp.int32)]).reshape(NC * NS, CPT, CHUNK)
    cols3 = jnp.concatenate(
        [col, jnp.zeros((n_pad,), jnp.int32)]).reshape(NC * NS, CPT, CHUNK)
    x_lo = x[:, :DH]
    x_hi = x[:, DH:]
    zsum = jnp.zeros((ROWS_PER_TILE, DH), jnp.float32)
    zdeg = jnp.zeros((ROWS_PER_TILE, DEG_W), jnp.float32)
    onescol = jnp.zeros((CHUNK, DEG_W), jnp.float32).at[:, 0].set(1.0)

    sum_lo, sum_hi, deg_partial = _sc_segment_sum(
        x_lo, x_hi, rows3, cols3, zsum, zdeg, onescol)
    wt = W.astype(jnp.float32).T
    return _tc_combine(x, wt, sum_lo, sum_hi, deg_partial).astype(jnp.float64)
